# initial kernel scaffold (unmeasured)
import jax
import jax.numpy as jnp
from jax import lax
from jax.experimental import pallas as pl
from jax.experimental.pallas import tpu as pltpu

N_RING = 8
COLS = 1024
M_HALF = 2048


def _ring_pos(yi, zi):
    return jnp.where(yi == 0, zi, 7 - zi)


def _ring_coords(q):
    y = jnp.where(q <= 3, 0, 1)
    z = jnp.where(q <= 3, q, 7 - q)
    return y, z


def _local_gemm(x_shard, dy_cols):
    m, k = x_shard.shape
    n = dy_cols.shape[1]
    tk = 512

    def body(x_ref, dy_ref, p_ref):
        p_ref[...] = lax.dot_general(
            x_ref[...], dy_ref[...],
            dimension_numbers=(((0,), (0,)), ((), ())),
            preferred_element_type=jnp.float32,
        )

    return pl.pallas_call(
        body,
        grid=(k // tk,),
        in_specs=[
            pl.BlockSpec((m, tk), lambda i: (0, i)),
            pl.BlockSpec((m, n), lambda i: (0, 0)),
        ],
        out_specs=pl.BlockSpec((tk, n), lambda i: (i, 0)),
        out_shape=jax.ShapeDtypeStruct((k, n), jnp.float32),
    )(x_shard, dy_cols)


def _reduce_x_and_allgather(p_partial):
    k, n = p_partial.shape

    def body(p_ref, out_ref, recvx_ref, x_sems,
             cw_send, cw_recv, ccw_send, ccw_recv):
        xi = lax.axis_index("x")
        yi = lax.axis_index("y")
        zi = lax.axis_index("z")
        p = _ring_pos(yi, zi)
        ry, rz = _ring_coords((p + 1) % N_RING)
        ly, lz = _ring_coords((p + N_RING - 1) % N_RING)

        barrier = pltpu.get_barrier_semaphore()
        for ty, tyy, tzz in ((1 - xi, yi, zi), (xi, ly, lz), (xi, ry, rz)):
            pl.semaphore_signal(
                barrier, inc=1,
                device_id=(ty, tyy, tzz),
                device_id_type=pl.DeviceIdType.MESH,
            )
        pl.semaphore_wait(barrier, 3)

        other = 1 - xi
        rdma_x = pltpu.make_async_remote_copy(
            src_ref=p_ref.at[pl.ds(other * M_HALF, M_HALF), :],
            dst_ref=recvx_ref,
            send_sem=x_sems.at[0],
            recv_sem=x_sems.at[1],
            device_id=(other, yi, zi),
            device_id_type=pl.DeviceIdType.MESH,
        )
        rdma_x.start()
        rdma_x.wait()

        out_ref[:, pl.ds(p * COLS, COLS)] = (
            p_ref[pl.ds(xi * M_HALF, M_HALF), :] + recvx_ref[...]
        )

        for h in range(4):
            o_cw = (p + N_RING - h) % N_RING
            rdma_cw = pltpu.make_async_remote_copy(
                src_ref=out_ref.at[:, pl.ds(o_cw * COLS, COLS)],
                dst_ref=out_ref.at[:, pl.ds(o_cw * COLS, COLS)],
                send_sem=cw_send.at[h],
                recv_sem=cw_recv.at[h],
                device_id=(xi, ry, rz),
                device_id_type=pl.DeviceIdType.MESH,
            )
            rdma_cw.start()
            if h < 3:
                o_ccw = (p + h) % N_RING
                rdma_ccw = pltpu.make_async_remote_copy(
                    src_ref=out_ref.at[:, pl.ds(o_ccw * COLS, COLS)],
                    dst_ref=out_ref.at[:, pl.ds(o_ccw * COLS, COLS)],
                    send_sem=ccw_send.at[h],
                    recv_sem=ccw_recv.at[h],
                    device_id=(xi, ly, lz),
                    device_id_type=pl.DeviceIdType.MESH,
                )
                rdma_ccw.start()
            rdma_cw.wait()
            if h < 3:
                rdma_ccw.wait()

    return pl.pallas_call(
        body,
        out_shape=jax.ShapeDtypeStruct((M_HALF, N_RING * n), jnp.float32),
        in_specs=[pl.BlockSpec(memory_space=pltpu.VMEM)],
        out_specs=pl.BlockSpec(memory_space=pltpu.VMEM),
        scratch_shapes=[
            pltpu.VMEM((M_HALF, n), jnp.float32),
            pltpu.SemaphoreType.DMA((2,)),
            pltpu.SemaphoreType.DMA((4,)),
            pltpu.SemaphoreType.DMA((4,)),
            pltpu.SemaphoreType.DMA((3,)),
            pltpu.SemaphoreType.DMA((3,)),
        ],
        compiler_params=pltpu.CompilerParams(collective_id=0),
    )(p_partial)


def kernel(x, dy):
    yi = lax.axis_index("y")
    zi = lax.axis_index("z")
    p = _ring_pos(yi, zi)
    dy_cols = lax.dynamic_slice_in_dim(dy, p * COLS, COLS, axis=1)
    partial = _local_gemm(x, dy_cols)
    return _reduce_x_and_allgather(partial)


# baseline (device time: 576511 ns/iter reference)
import jax
import jax.numpy as jnp
from jax import lax
from jax.experimental import pallas as pl
from jax.experimental.pallas import tpu as pltpu

N_RING = 8
COLS = 1024
M_HALF = 2048


def _ring_pos(yi, zi):
    return jnp.where(yi == 0, zi, 7 - zi)


def _ring_coords(q):
    y = jnp.where(q <= 3, 0, 1)
    z = jnp.where(q <= 3, q, 7 - q)
    return y, z


def _local_gemm(x_shard, dy_cols):
    m, k = x_shard.shape
    n = dy_cols.shape[1]
    tk = 512

    def body(x_ref, dy_ref, p_ref):
        p_ref[...] = lax.dot_general(
            x_ref[...], dy_ref[...],
            dimension_numbers=(((0,), (0,)), ((), ())),
            preferred_element_type=jnp.float32,
        )

    return pl.pallas_call(
        body,
        grid=(k // tk,),
        in_specs=[
            pl.BlockSpec((m, tk), lambda i: (0, i)),
            pl.BlockSpec((m, n), lambda i: (0, 0)),
        ],
        out_specs=pl.BlockSpec((tk, n), lambda i: (i, 0)),
        out_shape=jax.ShapeDtypeStruct((k, n), jnp.float32),
        compiler_params=pltpu.CompilerParams(
            vmem_limit_bytes=64 * 1024 * 1024,
        ),
    )(x_shard, dy_cols)


def _reduce_x_and_allgather(p_partial):
    k, n = p_partial.shape

    def body(p_ref, out_ref, own_ref, recvx_ref, copy_sems, x_sems,
             cw_send, cw_recv, ccw_send, ccw_recv):
        xi = lax.axis_index("x")
        yi = lax.axis_index("y")
        zi = lax.axis_index("z")
        p = _ring_pos(yi, zi)
        ry, rz = _ring_coords((p + 1) % N_RING)
        ly, lz = _ring_coords((p + N_RING - 1) % N_RING)

        load = pltpu.make_async_copy(
            p_ref.at[pl.ds(xi * M_HALF, M_HALF), :],
            own_ref,
            copy_sems.at[0],
        )
        load.start()

        barrier = pltpu.get_barrier_semaphore()
        for ty, tyy, tzz in ((1 - xi, yi, zi), (xi, ly, lz), (xi, ry, rz)):
            pl.semaphore_signal(
                barrier, inc=1,
                device_id=(ty, tyy, tzz),
                device_id_type=pl.DeviceIdType.MESH,
            )
        pl.semaphore_wait(barrier, 3)

        other = 1 - xi
        rdma_x = pltpu.make_async_remote_copy(
            src_ref=p_ref.at[pl.ds(other * M_HALF, M_HALF), :],
            dst_ref=recvx_ref,
            send_sem=x_sems.at[0],
            recv_sem=x_sems.at[1],
            device_id=(other, yi, zi),
            device_id_type=pl.DeviceIdType.MESH,
        )
        rdma_x.start()
        rdma_x.wait()
        load.wait()

        own_ref[...] = own_ref[...] + recvx_ref[...]
        store = pltpu.make_async_copy(
            own_ref,
            out_ref.at[:, pl.ds(p * COLS, COLS)],
            copy_sems.at[1],
        )
        store.start()
        store.wait()

        for h in range(4):
            o_cw = (p + N_RING - h) % N_RING
            rdma_cw = pltpu.make_async_remote_copy(
                src_ref=out_ref.at[:, pl.ds(o_cw * COLS, COLS)],
                dst_ref=out_ref.at[:, pl.ds(o_cw * COLS, COLS)],
                send_sem=cw_send.at[h],
                recv_sem=cw_recv.at[h],
                device_id=(xi, ry, rz),
                device_id_type=pl.DeviceIdType.MESH,
            )
            rdma_cw.start()
            if h < 3:
                o_ccw = (p + h) % N_RING
                rdma_ccw = pltpu.make_async_remote_copy(
                    src_ref=out_ref.at[:, pl.ds(o_ccw * COLS, COLS)],
                    dst_ref=out_ref.at[:, pl.ds(o_ccw * COLS, COLS)],
                    send_sem=ccw_send.at[h],
                    recv_sem=ccw_recv.at[h],
                    device_id=(xi, ly, lz),
                    device_id_type=pl.DeviceIdType.MESH,
                )
                rdma_ccw.start()
            rdma_cw.wait()
            if h < 3:
                rdma_ccw.wait()

    return pl.pallas_call(
        body,
        out_shape=jax.ShapeDtypeStruct((M_HALF, N_RING * n), jnp.float32),
        in_specs=[pl.BlockSpec(memory_space=pl.ANY)],
        out_specs=pl.BlockSpec(memory_space=pl.ANY),
        scratch_shapes=[
            pltpu.VMEM((M_HALF, n), jnp.float32),
            pltpu.VMEM((M_HALF, n), jnp.float32),
            pltpu.SemaphoreType.DMA((2,)),
            pltpu.SemaphoreType.DMA((2,)),
            pltpu.SemaphoreType.DMA((4,)),
            pltpu.SemaphoreType.DMA((4,)),
            pltpu.SemaphoreType.DMA((3,)),
            pltpu.SemaphoreType.DMA((3,)),
        ],
        compiler_params=pltpu.CompilerParams(
            collective_id=0,
            vmem_limit_bytes=48 * 1024 * 1024,
        ),
    )(p_partial)


def kernel(x, dy):
    yi = lax.axis_index("y")
    zi = lax.axis_index("z")
    p = _ring_pos(yi, zi)
    dy_cols = lax.dynamic_slice_in_dim(dy, p * COLS, COLS, axis=1)
    partial = _local_gemm(x, dy_cols)
    return _reduce_x_and_allgather(partial)


# device time: 483789 ns/iter; 1.1917x vs baseline; 1.1917x over previous
import jax
import jax.numpy as jnp
from jax import lax
from jax.experimental import pallas as pl
from jax.experimental.pallas import tpu as pltpu

N_RING = 8
COLS = 1024
M_HALF = 2048
N_SUB = 4
SUB = COLS // N_SUB
TK = 512
N_KT = 8
N_CW = 4
N_CCW = 3


def _ring_pos(yi, zi):
    return jnp.where(yi == 0, zi, 7 - zi)


def _ring_coords(q):
    y = jnp.where(q <= 3, 0, 1)
    z = jnp.where(q <= 3, q, 7 - q)
    return y, z


def _fused(x_shard, dy_cols):
    m, k = x_shard.shape

    def body(x_ref, dy_ref, out_ref, pbuf, recvx, store_sems,
             xs_send, xs_recv, cw_send, cw_recv, ccw_send, ccw_recv):
        c = pl.program_id(0)
        kt = pl.program_id(1)
        xi = lax.axis_index("x")
        yi = lax.axis_index("y")
        zi = lax.axis_index("z")
        p = _ring_pos(yi, zi)
        ry, rz = _ring_coords((p + 1) % N_RING)
        ly, lz = _ring_coords((p + N_RING - 1) % N_RING)
        other = 1 - xi

        @pl.when((c == 0) & (kt == 0))
        def _():
            barrier = pltpu.get_barrier_semaphore()
            for tgt in ((other, yi, zi), (xi, ly, lz), (xi, ry, rz)):
                pl.semaphore_signal(
                    barrier, inc=1,
                    device_id=tgt,
                    device_id_type=pl.DeviceIdType.MESH,
                )
            pl.semaphore_wait(barrier, 3)

        pbuf[c % 2, pl.ds(kt * TK, TK), :] = lax.dot_general(
            x_ref[...], dy_ref[...],
            dimension_numbers=(((0,), (0,)), ((), ())),
            preferred_element_type=jnp.float32,
        )

        def xexch(cc):
            return pltpu.make_async_remote_copy(
                src_ref=pbuf.at[cc % 2, pl.ds(other * M_HALF, M_HALF), :],
                dst_ref=recvx.at[cc],
                send_sem=xs_send.at[cc],
                recv_sem=xs_recv.at[cc],
                device_id=(other, yi, zi),
                device_id_type=pl.DeviceIdType.MESH,
            )

        def cw(cc, h):
            o = (p + N_RING - h) % N_RING
            sl = pl.ds(o * COLS + cc * SUB, SUB)
            return pltpu.make_async_remote_copy(
                src_ref=out_ref.at[:, sl],
                dst_ref=out_ref.at[:, sl],
                send_sem=cw_send.at[cc * N_CW + h],
                recv_sem=cw_recv.at[cc * N_CW + h],
                device_id=(xi, ry, rz),
                device_id_type=pl.DeviceIdType.MESH,
            )

        def ccw(cc, h):
            o = (p + h) % N_RING
            sl = pl.ds(o * COLS + cc * SUB, SUB)
            return pltpu.make_async_remote_copy(
                src_ref=out_ref.at[:, sl],
                dst_ref=out_ref.at[:, sl],
                send_sem=ccw_send.at[cc * N_CCW + h],
                recv_sem=ccw_recv.at[cc * N_CCW + h],
                device_id=(xi, ly, lz),
                device_id_type=pl.DeviceIdType.MESH,
            )

        def act_a(cc):
            xexch(cc).wait()
            recvx[cc] = recvx[cc] + pbuf[cc % 2, pl.ds(xi * M_HALF, M_HALF), :]
            st = pltpu.make_async_copy(
                recvx.at[cc],
                out_ref.at[:, pl.ds(p * COLS + cc * SUB, SUB)],
                store_sems.at[cc],
            )
            st.start()
            st.wait()
            cw(cc, 0).start()
            ccw(cc, 0).start()

        def fwd_cw(cc, h):
            cw(cc, h).wait()
            if h + 1 < N_CW:
                cw(cc, h + 1).start()

        def fwd_ccw(cc, h):
            ccw(cc, h).wait()
            if h + 1 < N_CCW:
                ccw(cc, h + 1).start()

        for cc in range(N_SUB):
            @pl.when((c == cc) & (kt == N_KT - 1))
            def _(cc=cc):
                xexch(cc).start()

        schedule = {
            (1, 3): [(act_a, 0)],
            (1, 7): [(fwd_cw, 0, 0), (fwd_ccw, 0, 0)],
            (2, 3): [(act_a, 1), (fwd_cw, 0, 1), (fwd_ccw, 0, 1)],
            (2, 7): [(fwd_cw, 1, 0), (fwd_ccw, 1, 0),
                     (fwd_cw, 0, 2), (fwd_ccw, 0, 2)],
            (3, 3): [(act_a, 2), (fwd_cw, 1, 1), (fwd_ccw, 1, 1),
                     (fwd_cw, 0, 3)],
            (3, 7): [(fwd_cw, 2, 0), (fwd_ccw, 2, 0),
                     (fwd_cw, 1, 2), (fwd_ccw, 1, 2),
                     (act_a, 3),
                     (fwd_cw, 2, 1), (fwd_ccw, 2, 1), (fwd_cw, 1, 3),
                     (fwd_cw, 3, 0), (fwd_ccw, 3, 0),
                     (fwd_cw, 2, 2), (fwd_ccw, 2, 2),
                     (fwd_cw, 3, 1), (fwd_ccw, 3, 1), (fwd_cw, 2, 3),
                     (fwd_cw, 3, 2), (fwd_ccw, 3, 2),
                     (fwd_cw, 3, 3)],
        }
        for (sc, skt), actions in schedule.items():
            @pl.when((c == sc) & (kt == skt))
            def _(actions=actions):
                for fn, *a in actions:
                    fn(*a)

    return pl.pallas_call(
        body,
        grid=(N_SUB, N_KT),
        in_specs=[
            pl.BlockSpec((m, TK), lambda c, kt: (0, kt)),
            pl.BlockSpec((m, SUB), lambda c, kt: (0, c)),
        ],
        out_specs=pl.BlockSpec(memory_space=pl.ANY),
        out_shape=jax.ShapeDtypeStruct((M_HALF, N_RING * COLS), jnp.float32),
        scratch_shapes=[
            pltpu.VMEM((2, k, SUB), jnp.float32),
            pltpu.VMEM((N_SUB, M_HALF, SUB), jnp.float32),
            pltpu.SemaphoreType.DMA((N_SUB,)),
            pltpu.SemaphoreType.DMA((N_SUB,)),
            pltpu.SemaphoreType.DMA((N_SUB,)),
            pltpu.SemaphoreType.DMA((N_SUB * N_CW,)),
            pltpu.SemaphoreType.DMA((N_SUB * N_CW,)),
            pltpu.SemaphoreType.DMA((N_SUB * N_CCW,)),
            pltpu.SemaphoreType.DMA((N_SUB * N_CCW,)),
        ],
        compiler_params=pltpu.CompilerParams(
            collective_id=0,
            dimension_semantics=("arbitrary", "arbitrary"),
            vmem_limit_bytes=56 * 1024 * 1024,
        ),
    )(x_shard, dy_cols)


def kernel(x, dy):
    yi = lax.axis_index("y")
    zi = lax.axis_index("z")
    p = _ring_pos(yi, zi)
    dy_cols = lax.dynamic_slice_in_dim(dy, p * COLS, COLS, axis=1)
    return _fused(x, dy_cols)
